# Initial kernel scaffold; baseline (speedup 1.0000x reference)
import functools

import jax
import jax.numpy as jnp
from jax import lax
from jax.experimental import pallas as pl
from jax.experimental.pallas import tpu as pltpu
from jax.experimental.pallas import tpu_sc as plsc

_D = 128
_SEG = 2048
_NC = 2
_NS = 16
_COLS = _D // _NC
_CHUNK = 256
_LANES = 16


def _segment_mean(z, inv):
    n_rows = z.shape[0]
    n_seg = n_rows // _SEG
    mesh = plsc.VectorSubcoreMesh(core_axis_name="c", subcore_axis_name="s")

    @functools.partial(
        pl.kernel,
        out_type=jax.ShapeDtypeStruct((n_seg, _D), jnp.float32),
        mesh=mesh,
        scratch_types=[
            pltpu.VMEM((_CHUNK, _COLS), jnp.float32),
            pltpu.VMEM((_COLS,), jnp.float32),
            pltpu.VMEM((_LANES,), jnp.float32),
        ],
    )
    def k(z_hbm, inv_hbm, out_hbm, buf, outv, invv):
        c = lax.axis_index("c")
        s = lax.axis_index("s")
        row0 = s * _SEG
        col0 = c * _COLS

        pltpu.sync_copy(inv_hbm, invv)

        def chunk_body(i, acc):
            pltpu.sync_copy(
                z_hbm.at[pl.ds(row0 + i * _CHUNK, _CHUNK), pl.ds(col0, _COLS)],
                buf,
            )

            def row_body(r, a):
                return tuple(
                    a[j] + buf[r, pl.ds(j * _LANES, _LANES)]
                    for j in range(_COLS // _LANES)
                )

            return lax.fori_loop(0, _CHUNK, row_body, acc)

        zero = jnp.zeros((_LANES,), jnp.float32)
        acc = lax.fori_loop(0, _SEG // _CHUNK, chunk_body,
                            (zero,) * (_COLS // _LANES))

        iv = invv[...]
        for j in range(_COLS // _LANES):
            outv[pl.ds(j * _LANES, _LANES)] = acc[j] * iv
        pltpu.sync_copy(outv, out_hbm.at[s, pl.ds(col0, _COLS)])

    return k(z, inv)


def kernel(Z_snd, splits):
    inv = jnp.full((_LANES,), 1.0, jnp.float32) / jnp.asarray(
        splits
    ).astype(jnp.float32)
    return _segment_mean(Z_snd, inv)


# speedup vs baseline: 3.3540x; 3.3540x over previous
import functools

import jax
import jax.numpy as jnp
from jax import lax
from jax.experimental import pallas as pl
from jax.experimental.pallas import tpu as pltpu
from jax.experimental.pallas import tpu_sc as plsc

_D = 128
_SEG = 2048
_NC = 2
_NS = 16
_COLS = _D // _NC
_CHUNK = 256
_LANES = 16


def _segment_mean(z, inv):
    n_rows = z.shape[0]
    n_seg = n_rows // _SEG
    mesh = plsc.VectorSubcoreMesh(core_axis_name="c", subcore_axis_name="s")

    @functools.partial(
        pl.kernel,
        out_type=jax.ShapeDtypeStruct((n_seg, _D), jnp.float32),
        mesh=mesh,
        scratch_types=[
            pltpu.VMEM((_CHUNK, _COLS), jnp.float32),
            pltpu.VMEM((_COLS,), jnp.float32),
            pltpu.VMEM((_LANES,), jnp.float32),
        ],
        compiler_params=pltpu.CompilerParams(use_tc_tiling_on_sc=False),
    )
    def k(z_hbm, inv_hbm, out_hbm, buf, outv, invv):
        c = lax.axis_index("c")
        s = lax.axis_index("s")
        row0 = s * _SEG
        col0 = c * _COLS

        pltpu.sync_copy(inv_hbm, invv)

        def chunk_body(i, acc):
            pltpu.sync_copy(
                z_hbm.at[pl.ds(row0 + i * _CHUNK, _CHUNK), pl.ds(col0, _COLS)],
                buf,
            )

            def row_body(r, a):
                return tuple(
                    a[j] + buf[r, pl.ds(j * _LANES, _LANES)]
                    for j in range(_COLS // _LANES)
                )

            return lax.fori_loop(0, _CHUNK, row_body, acc)

        zero = jnp.zeros((_LANES,), jnp.float32)
        acc = lax.fori_loop(0, _SEG // _CHUNK, chunk_body,
                            (zero,) * (_COLS // _LANES))

        iv = invv[...]
        for j in range(_COLS // _LANES):
            outv[pl.ds(j * _LANES, _LANES)] = acc[j] * iv
        pltpu.sync_copy(outv, out_hbm.at[s, pl.ds(col0, _COLS)])

    return k(z, inv)


def kernel(Z_snd, splits):
    inv = jnp.full((_LANES,), 1.0, jnp.float32) / jnp.asarray(
        splits
    ).astype(jnp.float32)
    return _segment_mean(Z_snd, inv)


# speedup vs baseline: 4.0367x; 1.2036x over previous
import functools

import jax
import jax.numpy as jnp
from jax import lax
from jax.experimental import pallas as pl
from jax.experimental.pallas import tpu as pltpu
from jax.experimental.pallas import tpu_sc as plsc

_D = 128
_SEG = 2048
_NC = 2
_NS = 16
_COLS = _D // _NC
_CHUNK = 256
_LANES = 16
_UNROLL = 8


def _segment_mean(z, inv):
    n_rows = z.shape[0]
    n_seg = n_rows // _SEG
    mesh = plsc.VectorSubcoreMesh(core_axis_name="c", subcore_axis_name="s")

    @functools.partial(
        pl.kernel,
        out_type=jax.ShapeDtypeStruct((n_seg, _D), jnp.float32),
        mesh=mesh,
        scratch_types=[
            pltpu.VMEM((_CHUNK, _COLS), jnp.float32),
            pltpu.VMEM((_CHUNK, _COLS), jnp.float32),
            pltpu.VMEM((_COLS,), jnp.float32),
            pltpu.VMEM((_LANES,), jnp.float32),
            pltpu.SemaphoreType.DMA,
            pltpu.SemaphoreType.DMA,
        ],
        compiler_params=pltpu.CompilerParams(use_tc_tiling_on_sc=False),
    )
    def k(z_hbm, inv_hbm, out_hbm, buf0, buf1, outv, invv, sem0, sem1):
        c = lax.axis_index("c")
        s = lax.axis_index("s")
        row0 = s * _SEG
        col0 = c * _COLS

        pltpu.sync_copy(inv_hbm, invv)

        bufs = (buf0, buf1)
        sems = (sem0, sem1)
        n_chunks = _SEG // _CHUNK

        def start(i):
            return pltpu.async_copy(
                z_hbm.at[pl.ds(row0 + i * _CHUNK, _CHUNK), pl.ds(col0, _COLS)],
                bufs[i % 2],
                sems[i % 2],
            )

        def accumulate(buf, acc):
            def row_body(r, a):
                out = []
                for j in range(_COLS // _LANES):
                    x = [buf[r * _UNROLL + u, pl.ds(j * _LANES, _LANES)]
                         for u in range(_UNROLL)]
                    t = [x[2 * p] + x[2 * p + 1] for p in range(_UNROLL // 2)]
                    out.append(a[j] + ((t[0] + t[1]) + (t[2] + t[3])))
                return tuple(out)

            return lax.fori_loop(0, _CHUNK // _UNROLL, row_body, acc)

        zero = jnp.zeros((_LANES,), jnp.float32)
        acc = (zero,) * (_COLS // _LANES)
        handles = [start(0), None]
        for i in range(n_chunks):
            if i + 1 < n_chunks:
                handles[(i + 1) % 2] = start(i + 1)
            handles[i % 2].wait()
            acc = accumulate(bufs[i % 2], acc)

        iv = invv[...]
        for j in range(_COLS // _LANES):
            outv[pl.ds(j * _LANES, _LANES)] = acc[j] * iv
        pltpu.sync_copy(outv, out_hbm.at[s, pl.ds(col0, _COLS)])

    return k(z, inv)


def kernel(Z_snd, splits):
    inv = jnp.full((_LANES,), 1.0, jnp.float32) / jnp.asarray(
        splits
    ).astype(jnp.float32)
    return _segment_mean(Z_snd, inv)


# speedup vs baseline: 4.0471x; 1.0026x over previous
import functools

import jax
import jax.numpy as jnp
from jax import lax
from jax.experimental import pallas as pl
from jax.experimental.pallas import tpu as pltpu
from jax.experimental.pallas import tpu_sc as plsc

_D = 128
_SEG = 2048
_NC = 2
_NS = 16
_COLS = _D // _NC
_CHUNK = 256
_LANES = 16
_UNROLL = 8


def _segment_mean(z, inv):
    n_rows = z.shape[0]
    n_seg = n_rows // _SEG
    mesh = plsc.VectorSubcoreMesh(core_axis_name="c", subcore_axis_name="s")

    @functools.partial(
        pl.kernel,
        out_type=jax.ShapeDtypeStruct((n_seg, _D), jnp.float32),
        mesh=mesh,
        scratch_types=[
            pltpu.VMEM((_CHUNK, _COLS), jnp.float32),
            pltpu.VMEM((_CHUNK, _COLS), jnp.float32),
            pltpu.VMEM((_COLS,), jnp.float32),
            pltpu.VMEM((_LANES,), jnp.float32),
            pltpu.SemaphoreType.DMA,
            pltpu.SemaphoreType.DMA,
        ],
        compiler_params=pltpu.CompilerParams(
            use_tc_tiling_on_sc=False,
            disable_bounds_checks=True,
            disable_semaphore_checks=True,
        ),
    )
    def k(z_hbm, inv_hbm, out_hbm, buf0, buf1, outv, invv, sem0, sem1):
        c = lax.axis_index("c")
        s = lax.axis_index("s")
        row0 = s * _SEG
        col0 = c * _COLS

        pltpu.sync_copy(inv_hbm, invv)

        bufs = (buf0, buf1)
        sems = (sem0, sem1)
        n_chunks = _SEG // _CHUNK

        def start(i):
            return pltpu.async_copy(
                z_hbm.at[pl.ds(row0 + i * _CHUNK, _CHUNK), pl.ds(col0, _COLS)],
                bufs[i % 2],
                sems[i % 2],
            )

        def accumulate(buf, acc):
            def row_body(r, a):
                out = []
                for j in range(_COLS // _LANES):
                    x = [buf[r * _UNROLL + u, pl.ds(j * _LANES, _LANES)]
                         for u in range(_UNROLL)]
                    t = [x[2 * p] + x[2 * p + 1] for p in range(_UNROLL // 2)]
                    out.append(a[j] + ((t[0] + t[1]) + (t[2] + t[3])))
                return tuple(out)

            return lax.fori_loop(0, _CHUNK // _UNROLL, row_body, acc)

        zero = jnp.zeros((_LANES,), jnp.float32)
        acc = (zero,) * (_COLS // _LANES)
        handles = [start(0), None]
        for i in range(n_chunks):
            if i + 1 < n_chunks:
                handles[(i + 1) % 2] = start(i + 1)
            handles[i % 2].wait()
            acc = accumulate(bufs[i % 2], acc)

        iv = invv[...]
        for j in range(_COLS // _LANES):
            outv[pl.ds(j * _LANES, _LANES)] = acc[j] * iv
        pltpu.sync_copy(outv, out_hbm.at[s, pl.ds(col0, _COLS)])

    return k(z, inv)


def kernel(Z_snd, splits):
    inv = jnp.full((_LANES,), 1.0, jnp.float32) / jnp.asarray(
        splits
    ).astype(jnp.float32)
    return _segment_mean(Z_snd, inv)
